# Initial kernel scaffold; baseline (speedup 1.0000x reference)
import jax
import jax.numpy as jnp
from jax.experimental import pallas as pl


def kernel(atomic_numbers, pos, edge_index, batch_ids, emb_table, W_rbf, W_edge, W_m1, W_m2, W_gate, W_am, W_h, W_e, W_out):
    raise NotImplementedError("write your pallas kernel here")


# speedup vs baseline: 1.2198x; 1.2198x over previous
import functools

import jax
import jax.numpy as jnp
from jax import lax
from jax.experimental import pallas as pl
from jax.experimental.pallas import tpu as pltpu
from jax.experimental.pallas import tpu_sc as plsc

N = 10000
E = 160000
NG = 64
NUM_RADIAL = 128
EMB_RBF = 16
D_ATOM = 256
D_EDGE = 512
N_BLOCKS = 4
CUTOFF = 12.0

TA = 256
NT = 40
NPAD = TA * NT
EP = 163840
TE = 1024
CE = 512
NW = 32
INV_SQRT2 = 0.7071067811865475


def _sc_gather(table, idx, chunk):
    V, D = table.shape
    B = idx.shape[0]
    per_w = B // NW
    n_iter = per_w // chunk
    assert per_w % chunk == 0 and B % NW == 0 and per_w % 8 == 0

    def body(table_hbm, idx_hbm, out_hbm, idx_v, rows_v, sem):
        wid = lax.axis_index("s") * 2 + lax.axis_index("c")
        base = wid * per_w

        def step(j, carry):
            off = base + j * chunk
            pltpu.sync_copy(idx_hbm.at[pl.ds(off, chunk)], idx_v)
            pltpu.async_copy(table_hbm.at[idx_v], rows_v, sem).wait()
            pltpu.sync_copy(rows_v, out_hbm.at[pl.ds(off, chunk)])
            return carry

        lax.fori_loop(0, n_iter, step, 0)

    mesh = plsc.VectorSubcoreMesh(core_axis_name="c", subcore_axis_name="s")
    fn = pl.kernel(
        body,
        out_type=jax.ShapeDtypeStruct((B, D), table.dtype),
        mesh=mesh,
        scratch_types=[
            pltpu.VMEM((chunk,), jnp.int32),
            pltpu.VMEM((chunk, D), table.dtype),
            pltpu.SemaphoreType.DMA,
        ],
    )
    return fn(table, idx)


def _edge_init_body(hs_ref, hd_ref, wr_ref, ws_ref, wd_ref,
                    we_ref, m_ref, re_ref):
    vec = hd_ref[:, D_ATOM:] - hs_ref[:, D_ATOM:]
    d2 = jnp.sum(vec * vec, axis=1, keepdims=True)
    dist = jnp.sqrt(d2 + 1e-12) + 1e-6
    n = lax.broadcasted_iota(jnp.int32, (1, NUM_RADIAL), 1).astype(jnp.float32) + 1.0
    rbf = jnp.sqrt(2.0 / CUTOFF) * jnp.sin(n * (jnp.pi / CUTOFF) * dist) / dist
    u = jnp.clip(dist / CUTOFF, 0.0, 1.0)
    u5 = u * u * u * u * u
    env = 1.0 + (-21.0) * u5 + 35.0 * (u5 * u) + (-15.0) * (u5 * u * u)
    env = jnp.where(u < 1.0, env, 0.0)
    rbf = rbf * env
    re = jnp.dot(rbf, wr_ref[...], preferred_element_type=jnp.float32)
    z = (jnp.dot(hs_ref[:, :D_ATOM], ws_ref[...], preferred_element_type=jnp.float32)
         + jnp.dot(hd_ref[:, :D_ATOM], wd_ref[...], preferred_element_type=jnp.float32)
         + jnp.dot(re, we_ref[...], preferred_element_type=jnp.float32))
    m_ref[...] = z * jax.nn.sigmoid(z)
    re_ref[...] = re


def _edge_init(hs, hd, W_rbf, We_s, We_d, We_r):
    grid = (EP // TE,)
    return pl.pallas_call(
        _edge_init_body,
        grid=grid,
        in_specs=[
            pl.BlockSpec((TE, D_ATOM + 128), lambda i: (i, 0)),
            pl.BlockSpec((TE, D_ATOM + 128), lambda i: (i, 0)),
            pl.BlockSpec((NUM_RADIAL, EMB_RBF), lambda i: (0, 0)),
            pl.BlockSpec((D_ATOM, D_EDGE), lambda i: (0, 0)),
            pl.BlockSpec((D_ATOM, D_EDGE), lambda i: (0, 0)),
            pl.BlockSpec((EMB_RBF, D_EDGE), lambda i: (0, 0)),
        ],
        out_specs=[
            pl.BlockSpec((TE, D_EDGE), lambda i: (i, 0)),
            pl.BlockSpec((TE, EMB_RBF), lambda i: (i, 0)),
        ],
        out_shape=[
            jax.ShapeDtypeStruct((EP, D_EDGE), jnp.float32),
            jax.ShapeDtypeStruct((EP, EMB_RBF), jnp.float32),
        ],
    )(hs, hd, W_rbf, We_s, We_d, We_r)


def _edge_mix_body(m_ref, re_ref, wg_ref, w1_ref, w2_ref, wa_ref,
                   mmid_ref, amsg_ref):
    m0 = m_ref[...]
    gate = jnp.dot(re_ref[...], wg_ref[...], preferred_element_type=jnp.float32)
    z1 = jnp.dot(m0, w1_ref[...], preferred_element_type=jnp.float32)
    m2 = z1 * jax.nn.sigmoid(z1) * gate
    z2 = jnp.dot(m2, w2_ref[...], preferred_element_type=jnp.float32)
    m2 = z2 * jax.nn.sigmoid(z2)
    mm = (m0 + m2) * INV_SQRT2
    mmid_ref[...] = mm
    amsg_ref[...] = jnp.dot(mm, wa_ref[...], preferred_element_type=jnp.float32)


def _edge_mix(m, rbf_emb, Wgate, Wm1, Wm2, Wam):
    grid = (EP // TE,)
    return pl.pallas_call(
        _edge_mix_body,
        grid=grid,
        in_specs=[
            pl.BlockSpec((TE, D_EDGE), lambda i: (i, 0)),
            pl.BlockSpec((TE, EMB_RBF), lambda i: (i, 0)),
            pl.BlockSpec((EMB_RBF, D_EDGE), lambda i: (0, 0)),
            pl.BlockSpec((D_EDGE, D_EDGE), lambda i: (0, 0)),
            pl.BlockSpec((D_EDGE, D_EDGE), lambda i: (0, 0)),
            pl.BlockSpec((D_EDGE, D_ATOM), lambda i: (0, 0)),
        ],
        out_specs=[
            pl.BlockSpec((TE, D_EDGE), lambda i: (i, 0)),
            pl.BlockSpec((TE, D_ATOM), lambda i: (i, 0)),
        ],
        out_shape=[
            jax.ShapeDtypeStruct((EP, D_EDGE), jnp.float32),
            jax.ShapeDtypeStruct((EP, D_ATOM), jnp.float32),
        ],
    )(m, rbf_emb, Wgate, Wm1, Wm2, Wam)


def _atom_body(starts_ref, amsg_hbm, dst_hbm, h_ref, wh_ref, out_ref,
               amsg_v, dst_v, agg_ref, sem1, sem2):
    t = pl.program_id(0)
    start = starts_ref[t]
    end = starts_ref[t + 1]
    c0 = start // CE
    nch = jnp.maximum(0, (end + CE - 1) // CE - c0)
    agg_ref[...] = jnp.zeros((TA, D_ATOM), jnp.float32)

    def step(j, carry):
        off = pl.multiple_of((c0 + j) * CE, CE)
        c1 = pltpu.make_async_copy(amsg_hbm.at[pl.ds(off, CE)], amsg_v, sem1)
        c2 = pltpu.make_async_copy(dst_hbm.at[pl.ds(off, CE)], dst_v, sem2)
        c1.start()
        c2.start()
        c1.wait()
        c2.wait()
        dstl = dst_v[...] - t * TA
        ids = lax.broadcasted_iota(jnp.int32, (TA, CE), 0)
        S = (ids == dstl[None, :]).astype(jnp.float32)
        agg_ref[...] += jnp.dot(S, amsg_v[...], preferred_element_type=jnp.float32)
        return carry

    lax.fori_loop(0, nch, step, 0)
    z = jnp.dot(agg_ref[...], wh_ref[...], preferred_element_type=jnp.float32)
    out_ref[...] = h_ref[...] + z * jax.nn.sigmoid(z)


def _atom_update(starts, amsg, dst_m, h, Wh):
    grid_spec = pltpu.PrefetchScalarGridSpec(
        num_scalar_prefetch=1,
        grid=(NT,),
        in_specs=[
            pl.BlockSpec(memory_space=pl.ANY),
            pl.BlockSpec(memory_space=pl.ANY),
            pl.BlockSpec((TA, D_ATOM), lambda t, starts: (t, 0)),
            pl.BlockSpec((D_ATOM, D_ATOM), lambda t, starts: (0, 0)),
        ],
        out_specs=pl.BlockSpec((TA, D_ATOM), lambda t, starts: (t, 0)),
        scratch_shapes=[
            pltpu.VMEM((CE, D_ATOM), jnp.float32),
            pltpu.VMEM((CE,), jnp.int32),
            pltpu.VMEM((TA, D_ATOM), jnp.float32),
            pltpu.SemaphoreType.DMA,
            pltpu.SemaphoreType.DMA,
        ],
    )
    return pl.pallas_call(
        _atom_body,
        grid_spec=grid_spec,
        out_shape=jax.ShapeDtypeStruct((NPAD, D_ATOM), jnp.float32),
    )(starts, amsg, dst_m, h, Wh)


def _edge_up_body(hs_ref, hd_ref, m_ref, ws_ref, wd_ref, wm_ref, out_ref):
    z = (jnp.dot(hs_ref[...], ws_ref[...], preferred_element_type=jnp.float32)
         + jnp.dot(hd_ref[...], wd_ref[...], preferred_element_type=jnp.float32)
         + jnp.dot(m_ref[...], wm_ref[...], preferred_element_type=jnp.float32))
    e = z * jax.nn.sigmoid(z)
    out_ref[...] = (m_ref[...] + e) * INV_SQRT2


def _edge_up(hs, hd, mmid, We_s, We_d, We_m):
    grid = (EP // TE,)
    return pl.pallas_call(
        _edge_up_body,
        grid=grid,
        in_specs=[
            pl.BlockSpec((TE, D_ATOM), lambda i: (i, 0)),
            pl.BlockSpec((TE, D_ATOM), lambda i: (i, 0)),
            pl.BlockSpec((TE, D_EDGE), lambda i: (i, 0)),
            pl.BlockSpec((D_ATOM, D_EDGE), lambda i: (0, 0)),
            pl.BlockSpec((D_ATOM, D_EDGE), lambda i: (0, 0)),
            pl.BlockSpec((D_EDGE, D_EDGE), lambda i: (0, 0)),
        ],
        out_specs=pl.BlockSpec((TE, D_EDGE), lambda i: (i, 0)),
        out_shape=jax.ShapeDtypeStruct((EP, D_EDGE), jnp.float32),
    )(hs, hd, mmid, We_s, We_d, We_m)


def _energy_body(h_ref, b_ref, w_ref, out_ref):
    t = pl.program_id(0)

    @pl.when(t == 0)
    def _():
        out_ref[...] = jnp.zeros_like(out_ref)

    e = jnp.dot(h_ref[...], w_ref[...], preferred_element_type=jnp.float32)
    bids = b_ref[0, 0, :]
    gids = lax.broadcasted_iota(jnp.int32, (NG, TA), 0)
    S = (gids == bids[None, :]).astype(jnp.float32)
    out_ref[...] += jnp.dot(S, e, preferred_element_type=jnp.float32)


def _energy(h, bids3, Wout_pad):
    return pl.pallas_call(
        _energy_body,
        grid=(NT,),
        in_specs=[
            pl.BlockSpec((TA, D_ATOM), lambda t: (t, 0)),
            pl.BlockSpec((1, 1, TA), lambda t: (t, 0, 0)),
            pl.BlockSpec((D_ATOM, 128), lambda t: (0, 0)),
        ],
        out_specs=pl.BlockSpec((NG, 128), lambda t: (0, 0)),
        out_shape=jax.ShapeDtypeStruct((NG, 128), jnp.float32),
    )(h, bids3, Wout_pad)


def kernel(atomic_numbers, pos, edge_index, batch_ids, emb_table, W_rbf,
           W_edge, W_m1, W_m2, W_gate, W_am, W_h, W_e, W_out):
    src = edge_index[0].astype(jnp.int32)
    dst = edge_index[1].astype(jnp.int32)
    perm = jnp.argsort(dst)
    dsts = dst[perm]
    srcs = src[perm]
    pad_e = EP - E
    zpad = jnp.zeros((pad_e,), jnp.int32)
    src_g = jnp.concatenate([srcs, zpad])
    dst_g = jnp.concatenate([dsts, zpad])
    dst_m = jnp.concatenate([dsts, jnp.full((pad_e,), 1 << 20, jnp.int32)])
    starts = jnp.searchsorted(
        dsts, jnp.arange(NT + 1, dtype=jnp.int32) * TA).astype(jnp.int32)

    an_pad = jnp.concatenate(
        [atomic_numbers.astype(jnp.int32), jnp.zeros((NPAD - N,), jnp.int32)])
    pos_pad = jnp.zeros((NPAD, 128), jnp.float32).at[:N, :3].set(pos)

    h = _sc_gather(emb_table, an_pad, 320)
    hp = jnp.concatenate([h, pos_pad], axis=1)
    hps = _sc_gather(hp, src_g, 256)
    hpd = _sc_gather(hp, dst_g, 256)

    m, rbf_emb = _edge_init(hps, hpd, W_rbf,
                            W_edge[:D_ATOM], W_edge[D_ATOM:2 * D_ATOM],
                            W_edge[2 * D_ATOM:])

    for i in range(N_BLOCKS):
        mmid, amsg = _edge_mix(m, rbf_emb, W_gate[i], W_m1[i], W_m2[i], W_am[i])
        h = _atom_update(starts, amsg, dst_m, h, W_h[i])
        hs = _sc_gather(h, src_g, 256)
        hd = _sc_gather(h, dst_g, 256)
        m = _edge_up(hs, hd, mmid, W_e[i][:D_ATOM],
                     W_e[i][D_ATOM:2 * D_ATOM], W_e[i][2 * D_ATOM:])

    bids3 = jnp.concatenate(
        [batch_ids.astype(jnp.int32),
         jnp.full((NPAD - N,), NG, jnp.int32)]).reshape(NT, 1, TA)
    wout_pad = jnp.zeros((D_ATOM, 128), jnp.float32).at[:, :1].set(W_out)
    energy = _energy(h, bids3, wout_pad)
    return energy[:, 0]
